# baseline (speedup 1.0000x reference)
import functools

import jax
import jax.numpy as jnp
from jax import lax
from jax.experimental import pallas as pl
from jax.experimental.pallas import tpu as pltpu
from jax.experimental.pallas import tpu_sc as plsc

L = 16
NW = 32
D = 128
BL = 128


def _sc_embed(emb_t, ids_t, cat_table, subcat_table, *, n_b, n_s, feat):
    keep = feat - 2
    out_w = keep + 2 * D
    n_cat = cat_table.size // D
    n_sub = subcat_table.size // D
    mesh = plsc.VectorSubcoreMesh(core_axis_name="c", subcore_axis_name="s")

    @functools.partial(
        pl.kernel,
        out_type=jax.ShapeDtypeStruct((n_s, out_w, n_b), jnp.float32),
        mesh=mesh,
        scratch_types=[
            pltpu.VMEM((n_cat * D,), jnp.float32),
            pltpu.VMEM((n_sub * D,), jnp.float32),
            pltpu.VMEM((2, BL), jnp.float32),
            pltpu.VMEM((2, BL), jnp.float32),
            pltpu.VMEM((2 * D, BL), jnp.float32),
            pltpu.VMEM((2 * D, BL), jnp.float32),
            pltpu.SemaphoreType.DMA,
            pltpu.SemaphoreType.DMA,
            pltpu.SemaphoreType.DMA,
            pltpu.SemaphoreType.DMA,
            pltpu.SemaphoreType.DMA,
            pltpu.SemaphoreType.DMA,
        ],
        compiler_params=pltpu.CompilerParams(
            use_tc_tiling_on_sc=True, needs_layout_passes=False),
    )
    def body(emb_hbm, ids_hbm, cat_hbm, sub_hbm, out_hbm,
             tc_v, ts_v, in0, in1, bk0, bk1,
             si0, si1, sk0, sk1, sw0, sw1):
        inb, bkb = (in0, in1), (bk0, bk1)
        si, sk, sw = (si0, si1), (sk0, sk1), (sw0, sw1)
        wid = lax.axis_index("s") * 2 + lax.axis_index("c")
        b0 = wid * BL

        tcopy1 = pltpu.make_async_copy(cat_hbm, tc_v, sk[0])
        tcopy2 = pltpu.make_async_copy(sub_hbm, ts_v, sk[1])
        tcopy1.start()
        tcopy2.start()

        def in_copy(j, b):
            return pltpu.make_async_copy(
                ids_hbm.at[j, :, pl.ds(b0, BL)], inb[b], si[b])

        def keep_copy(j, b):
            return pltpu.make_async_copy(
                emb_hbm.at[j, pl.ds(0, keep), pl.ds(b0, BL)],
                out_hbm.at[j, pl.ds(0, keep), pl.ds(b0, BL)], sk[b])

        def w_copy(j, b):
            return pltpu.make_async_copy(
                bkb[b], out_hbm.at[j, pl.ds(keep, 2 * D), pl.ds(b0, BL)],
                sw[b])

        in_copy(0, 0).start()
        tcopy1.wait()
        tcopy2.wait()

        @pl.loop(0, n_s, step=2)
        def _(u):
            for b in (0, 1):
                j = u + b

                @pl.when(j >= 2)
                def _():
                    w_copy(j - 2, b).wait()
                    keep_copy(j - 2, b).wait()

                @pl.when(j < n_s - 1)
                def _():
                    in_copy(j + 1, 1 - b).start()

                keep_copy(j, b).start()
                in_copy(j, b).wait()

                for g in range(BL // L):
                    lanes = pl.ds(g * L, L)
                    idc = inb[b][0, lanes].astype(jnp.int32)
                    ids_ = inb[b][1, lanes].astype(jnp.int32)
                    U = 8

                    @pl.loop(0, D, step=U, init_carry=(idc, ids_))
                    def _(c, carry):
                        ac, as_ = carry
                        vc = [plsc.load_gather(tc_v, [ac + k * n_cat])
                              for k in range(U)]
                        vs = [plsc.load_gather(ts_v, [as_ + k * n_sub])
                              for k in range(U)]
                        for k in range(U):
                            bkb[b][c + k, lanes] = vc[k]
                        for k in range(U):
                            bkb[b][D + c + k, lanes] = vs[k]
                        return ac + U * n_cat, as_ + U * n_sub

                w_copy(j, b).start()

        w_copy(n_s - 2, 0).wait()
        keep_copy(n_s - 2, 0).wait()
        w_copy(n_s - 1, 1).wait()
        keep_copy(n_s - 1, 1).wait()

    return body(emb_t, ids_t, cat_table, subcat_table)


def kernel(embeddings, cat_table, subcat_table):
    n_b, n_s, feat = embeddings.shape
    emb_t = jnp.transpose(embeddings, (1, 2, 0))
    ids_t = jnp.transpose(embeddings[..., feat - 2:], (1, 2, 0))
    out_t = _sc_embed(emb_t, ids_t,
                      cat_table.T.reshape(-1), subcat_table.T.reshape(-1),
                      n_b=n_b, n_s=n_s, feat=feat)
    return jnp.transpose(out_t, (2, 0, 1))


# speedup vs baseline: 10.8403x; 10.8403x over previous
import functools

import jax
import jax.numpy as jnp
from jax import lax
from jax.experimental import pallas as pl
from jax.experimental.pallas import tpu as pltpu
from jax.experimental.pallas import tpu_sc as plsc

L = 16
NW = 32
D = 128
BL = 128


def _sc_embed(emb_t, ids_t, cat_table, subcat_table, *, n_b, n_s, feat):
    keep = feat - 2
    out_w = keep + 2 * D
    n_cat = cat_table.size // D
    n_sub = subcat_table.size // D
    mesh = plsc.VectorSubcoreMesh(core_axis_name="c", subcore_axis_name="s")

    @functools.partial(
        pl.kernel,
        out_type=jax.ShapeDtypeStruct((n_s, out_w, n_b), jnp.float32),
        mesh=mesh,
        scratch_types=[
            pltpu.VMEM((n_cat * D,), jnp.float32),
            pltpu.VMEM((n_sub * D,), jnp.float32),
            pltpu.VMEM((2, BL), jnp.float32),
            pltpu.VMEM((2, BL), jnp.float32),
            pltpu.VMEM((out_w, BL), jnp.float32),
            pltpu.VMEM((out_w, BL), jnp.float32),
            pltpu.SemaphoreType.DMA,
            pltpu.SemaphoreType.DMA,
            pltpu.SemaphoreType.DMA,
            pltpu.SemaphoreType.DMA,
            pltpu.SemaphoreType.DMA,
            pltpu.SemaphoreType.DMA,
        ],
        compiler_params=pltpu.CompilerParams(
            use_tc_tiling_on_sc=True, needs_layout_passes=False),
    )
    def body(emb_hbm, ids_hbm, cat_hbm, sub_hbm, out_hbm,
             tc_v, ts_v, in0, in1, bk0, bk1,
             si0, si1, sk0, sk1, sw0, sw1):
        inb, bkb = (in0, in1), (bk0, bk1)
        si, sk, sw = (si0, si1), (sk0, sk1), (sw0, sw1)
        wid = lax.axis_index("s") * 2 + lax.axis_index("c")
        b0 = wid * BL

        tcopy1 = pltpu.make_async_copy(cat_hbm, tc_v, sk[0])
        tcopy2 = pltpu.make_async_copy(sub_hbm, ts_v, sk[1])
        tcopy1.start()
        tcopy2.start()

        def in_copy(j, b):
            return pltpu.make_async_copy(
                ids_hbm.at[j, :, pl.ds(b0, BL)], inb[b], si[b])

        def keep_copy(j, b):
            return pltpu.make_async_copy(
                emb_hbm.at[j, pl.ds(0, keep), pl.ds(b0, BL)],
                bkb[b].at[pl.ds(0, keep)], sk[b])

        def w_copy(j, b):
            return pltpu.make_async_copy(
                bkb[b], out_hbm.at[j, :, pl.ds(b0, BL)], sw[b])

        in_copy(0, 0).start()
        tcopy1.wait()
        tcopy2.wait()

        @pl.loop(0, n_s, step=2)
        def _(u):
            for b in (0, 1):
                j = u + b

                @pl.when(j >= 2)
                def _():
                    w_copy(j - 2, b).wait()

                @pl.when(j < n_s - 1)
                def _():
                    in_copy(j + 1, 1 - b).start()

                keep_copy(j, b).start()
                in_copy(j, b).wait()

                for g in range(BL // L):
                    lanes = pl.ds(g * L, L)
                    idc = inb[b][0, lanes].astype(jnp.int32)
                    ids_ = inb[b][1, lanes].astype(jnp.int32)
                    U = 8

                    @pl.loop(0, D, step=U, init_carry=(idc, ids_))
                    def _(c, carry):
                        ac, as_ = carry
                        vc = [plsc.load_gather(tc_v, [ac + k * n_cat])
                              for k in range(U)]
                        vs = [plsc.load_gather(ts_v, [as_ + k * n_sub])
                              for k in range(U)]
                        for k in range(U):
                            bkb[b][keep + c + k, lanes] = vc[k]
                        for k in range(U):
                            bkb[b][keep + D + c + k, lanes] = vs[k]
                        return ac + U * n_cat, as_ + U * n_sub

                keep_copy(j, b).wait()
                w_copy(j, b).start()

        w_copy(n_s - 2, 0).wait()
        w_copy(n_s - 1, 1).wait()

    return body(emb_t, ids_t, cat_table, subcat_table)


def kernel(embeddings, cat_table, subcat_table):
    n_b, n_s, feat = embeddings.shape
    emb_t = jnp.transpose(embeddings, (1, 2, 0))
    ids_t = jnp.transpose(embeddings[..., feat - 2:], (1, 2, 0))
    out_t = _sc_embed(emb_t, ids_t,
                      cat_table.T.reshape(-1), subcat_table.T.reshape(-1),
                      n_b=n_b, n_s=n_s, feat=feat)
    return jnp.transpose(out_t, (2, 0, 1))
